# Initial kernel scaffold; baseline (speedup 1.0000x reference)
import jax
import jax.numpy as jnp
from jax.experimental import pallas as pl


def kernel(x, edge_index, batch, W_lin, b_lin, W_g, b_g):
    raise NotImplementedError("write your pallas kernel here")


# speedup vs baseline: 25.9233x; 25.9233x over previous
import functools

import jax
import jax.numpy as jnp
from jax import lax
from jax.experimental import pallas as pl
from jax.experimental.pallas import tpu as pltpu
from jax.experimental.pallas import tpu_sc as plsc

_N = 100000
_D = 16
_HD = 8
_E = 3200000
_NC = 2
_NS = 16
_NW = _NC * _NS
_EPD = 128
_R = 784
_E_PAD = _R * _NW * _EPD
_ROWS = _E_PAD // _EPD
_KI = 112
_G = 8
_NGROUP = _KI // _G
_NCHUNK = _R // _KI
_N_PAD = 100096
_RPS = _N_PAD // _NS


def _gs_body(h2_hbm, zeros_hbm, srclo_hbm, srchi_hbm, dst_hbm, out_hbm,
             srcbuf, dstbuf, m0, m1, sem0, sem1, agg):
    c = lax.axis_index("c")
    s = lax.axis_index("s")
    wid = c * _NS + s
    tb = wid * _R

    def fire(src_hbm, buf, sem, g):
        for i in range(_G):
            pltpu.async_copy(h2_hbm.at[srcbuf.at[g * _G + i]], buf.at[i],
                             sem)

    def drain(buf, sem, g):
        for i in range(_G):
            pltpu.make_async_copy(h2_hbm.at[srcbuf.at[g * _G + i]],
                                  buf.at[i], sem).wait()

    def scat(buf, g):
        for i in range(_G):
            pltpu.sync_copy(buf.at[i], agg.at[dstbuf.at[g * _G + i]],
                            add=True)

    for f, src_hbm in ((0, srclo_hbm), (1, srchi_hbm)):
        pltpu.sync_copy(zeros_hbm.at[pl.ds(s * _RPS, _RPS)],
                        agg.at[pl.ds(s * _RPS, _RPS)])
        plsc.subcore_barrier()

        @pl.loop(0, _NCHUNK)
        def _chunk(ci):
            row0 = tb + ci * _KI
            pltpu.sync_copy(src_hbm.at[pl.ds(row0, _KI)], srcbuf)
            pltpu.sync_copy(dst_hbm.at[pl.ds(row0, _KI)], dstbuf)

            fire(src_hbm, m0, sem0, 0)

            @pl.loop(0, _NGROUP, step=2)
            def _grp(g):
                fire(src_hbm, m1, sem1, g + 1)
                drain(m0, sem0, g)
                scat(m0, g)

                @pl.when(g + 2 < _NGROUP)
                def _():
                    fire(src_hbm, m0, sem0, g + 2)

                drain(m1, sem1, g + 1)
                scat(m1, g + 1)

        plsc.subcore_barrier()
        pltpu.sync_copy(agg.at[pl.ds(s * _RPS, _RPS)],
                        out_hbm.at[c, f, pl.ds(s * _RPS, _RPS)])


_gather_scatter = pl.kernel(
    _gs_body,
    out_type=jax.ShapeDtypeStruct((_NC, 2, _N_PAD, _HD), jnp.float32),
    mesh=plsc.VectorSubcoreMesh(core_axis_name="c", subcore_axis_name="s"),
    compiler_params=pltpu.CompilerParams(use_tc_tiling_on_sc=False),
    scratch_types=[
        pltpu.VMEM((_KI, _EPD), jnp.int32),
        pltpu.VMEM((_KI, _EPD), jnp.int32),
        pltpu.VMEM((_G, _EPD, _HD), jnp.float32),
        pltpu.VMEM((_G, _EPD, _HD), jnp.float32),
        pltpu.SemaphoreType.DMA,
        pltpu.SemaphoreType.DMA,
        pltpu.VMEM_SHARED((_N_PAD, _HD), jnp.float32),
    ],
)


_BR = 5000


def _dense1_body(x_ref, w_ref, b_ref, o_ref):
    o_ref[...] = jnp.dot(x_ref[...], w_ref[...],
                         preferred_element_type=jnp.float32) + b_ref[...]


def _dense2_body(act, p_ref, h_ref, w_ref, b_ref, o_ref):
    lo = p_ref[0, 0] + p_ref[1, 0]
    hi = p_ref[0, 1] + p_ref[1, 1]
    a = jnp.concatenate([lo, hi], axis=-1) + h_ref[...]
    o_ref[...] = act(jnp.dot(a, w_ref[...],
                             preferred_element_type=jnp.float32) + b_ref[...])


_linear = pl.pallas_call(
    _dense1_body,
    grid=(_N // _BR,),
    in_specs=[
        pl.BlockSpec((_BR, _D), lambda i: (i, 0)),
        pl.BlockSpec((_D, _D), lambda i: (0, 0)),
        pl.BlockSpec((1, _D), lambda i: (0, 0)),
    ],
    out_specs=pl.BlockSpec((_BR, _D), lambda i: (i, 0)),
    out_shape=jax.ShapeDtypeStruct((_N, _D), jnp.float32),
)


def _make_dense2(act):
    return pl.pallas_call(
        functools.partial(_dense2_body, act),
        grid=(_N // _BR,),
        in_specs=[
            pl.BlockSpec((_NC, 2, _BR, _HD), lambda i: (0, 0, i, 0)),
            pl.BlockSpec((_BR, _D), lambda i: (i, 0)),
            pl.BlockSpec((_D, _D), lambda i: (0, 0)),
            pl.BlockSpec((1, _D), lambda i: (0, 0)),
        ],
        out_specs=pl.BlockSpec((_BR, _D), lambda i: (i, 0)),
        out_shape=jax.ShapeDtypeStruct((_N, _D), jnp.float32),
    )


_dense2_relu = _make_dense2(jax.nn.relu)
_dense2_sigmoid = _make_dense2(jax.nn.sigmoid)


def kernel(x, edge_index, batch, W_lin, b_lin, W_g, b_g):
    del batch
    src = edge_index[0]
    dst = edge_index[1]
    pad = _E_PAD - _E
    srclo = jnp.concatenate(
        [src * 2, jnp.zeros((pad,), jnp.int32)]).reshape(_ROWS, _EPD)
    srchi = jnp.concatenate(
        [src * 2 + 1, jnp.zeros((pad,), jnp.int32)]).reshape(_ROWS, _EPD)
    dst_p = jnp.concatenate(
        [dst, jnp.full((pad,), _N, jnp.int32)]).reshape(_ROWS, _EPD)
    zeros = jnp.zeros((_N_PAD, _HD), jnp.float32)
    b_lin2 = b_lin.reshape(1, _D)
    b_g2 = b_g.reshape(1, _D)

    h = _linear(x, W_lin, b_lin2)
    for act_dense in (_dense2_relu, _dense2_relu, _dense2_sigmoid):
        h2 = h.reshape(2 * _N, _HD)
        p = _gather_scatter(h2, zeros, srclo, srchi, dst_p)
        h = act_dense(p, h, W_g, b_g2)
    return h
